# Initial kernel scaffold; baseline (speedup 1.0000x reference)
import functools

import jax
import jax.numpy as jnp
from jax import lax
from jax.experimental import pallas as pl
from jax.experimental.pallas import tpu as pltpu
from jax.experimental.pallas import tpu_sc as plsc

_NC = 2
_NS = 16
_NW = _NC * _NS
_K = 80
_CW = 16


def _seg_kernel(n, d, e, with_cnt):
    ew = e // _NW
    nchunk = ew // _K
    rt = n // _NS

    mesh = plsc.VectorSubcoreMesh(core_axis_name="c", subcore_axis_name="s")
    out_type = [jax.ShapeDtypeStruct((_NC, n, d), jnp.float32)]
    scratch = [
        pltpu.VMEM((_K,), jnp.int32),
        pltpu.VMEM((_K,), jnp.int32),
        pltpu.VMEM((_K, d), jnp.float32),
        pltpu.VMEM_SHARED((n, d), jnp.float32),
        pltpu.SemaphoreType.DMA,
    ]
    if with_cnt:
        out_type.append(jax.ShapeDtypeStruct((_NC, n, _CW), jnp.float32))
        scratch.insert(3, pltpu.VMEM((_K, _CW), jnp.float32))
        scratch.insert(4, pltpu.VMEM_SHARED((n, _CW), jnp.float32))

    def body(h_hbm, src_hbm, dst_hbm, zeros_hbm, *rest):
        if with_cnt:
            (ones_hbm, out_hbm, cnt_hbm,
             src_v, dst_v, rows_v, ones_v, acc_sh, cnt_sh, sem) = rest
        else:
            (out_hbm, src_v, dst_v, rows_v, acc_sh, sem) = rest
        c = lax.axis_index("c")
        s = lax.axis_index("s")
        wid = c * _NS + s
        r0 = s * rt
        pltpu.sync_copy(zeros_hbm.at[pl.ds(r0, rt)], acc_sh.at[pl.ds(r0, rt)])
        if with_cnt:
            pltpu.sync_copy(zeros_hbm.at[pl.ds(r0, rt), pl.ds(0, _CW)],
                            cnt_sh.at[pl.ds(r0, rt)])
            pltpu.sync_copy(ones_hbm, ones_v)
        plsc.subcore_barrier()

        base = wid * ew

        @pl.loop(0, nchunk)
        def _chunk(j):
            off = base + j * _K
            pltpu.sync_copy(src_hbm.at[pl.ds(off, _K)], src_v)
            pltpu.sync_copy(dst_hbm.at[pl.ds(off, _K)], dst_v)
            pltpu.async_copy(h_hbm.at[src_v], rows_v, sem).wait()
            pltpu.sync_copy(rows_v, acc_sh.at[dst_v], add=True)
            if with_cnt:
                pltpu.sync_copy(ones_v, cnt_sh.at[dst_v], add=True)

        plsc.subcore_barrier()
        pltpu.sync_copy(acc_sh.at[pl.ds(r0, rt)],
                        out_hbm.at[c, pl.ds(r0, rt)])
        if with_cnt:
            pltpu.sync_copy(cnt_sh.at[pl.ds(r0, rt)],
                            cnt_hbm.at[c, pl.ds(r0, rt)])

    return pl.kernel(body, out_type=out_type, mesh=mesh,
                     scratch_types=scratch)


def _tc_layer(n, d, mode):
    br = 400
    grid = (n // br,)

    def body(*refs):
        if mode == 1:
            p_ref, cnt_ref, hin_ref, wl_ref, b_ref, wr_ref, wsk_ref, a_ref, out_ref = refs
        else:
            p_ref, cnt_ref, hin_ref, wl_ref, b_ref, wr_ref, a_ref, out_ref = refs
        cnt = cnt_ref[0, :, 0:1] + cnt_ref[1, :, 0:1]
        inv = 1.0 / jnp.maximum(cnt, 1.0)
        agg = (p_ref[0] + p_ref[1]) * inv
        hin = hin_ref[...]
        dn = (((1,), (1,)), ((), ()))
        h = lax.dot_general(agg, wl_ref[...], dn,
                            precision=lax.Precision.HIGHEST,
                            preferred_element_type=jnp.float32)
        h = h + b_ref[...] + lax.dot_general(
            hin, wr_ref[...], dn, precision=lax.Precision.HIGHEST,
            preferred_element_type=jnp.float32)
        av = a_ref[0, 0]
        h = jnp.where(h >= 0, h, av * h)
        if mode == 1:
            h = h + lax.dot_general(hin, wsk_ref[...], dn,
                                    precision=lax.Precision.HIGHEST,
                                    preferred_element_type=jnp.float32)
        elif mode == 2:
            h = h + hin
        out_ref[...] = h

    in_specs = [
        pl.BlockSpec((_NC, br, d), lambda i: (0, i, 0)),
        pl.BlockSpec((_NC, br, _CW), lambda i: (0, i, 0)),
        pl.BlockSpec((br, d), lambda i: (i, 0)),
        pl.BlockSpec((d, d), lambda i: (0, 0)),
        pl.BlockSpec((1, d), lambda i: (0, 0)),
        pl.BlockSpec((d, d), lambda i: (0, 0)),
    ]
    if mode == 1:
        in_specs.append(pl.BlockSpec((d, d), lambda i: (0, 0)))
    in_specs.append(pl.BlockSpec(memory_space=pltpu.SMEM))

    return pl.pallas_call(
        body,
        grid=grid,
        in_specs=in_specs,
        out_specs=pl.BlockSpec((br, d), lambda i: (i, 0)),
        out_shape=jax.ShapeDtypeStruct((n, d), jnp.float32),
    )


def kernel(x, edge_index, W_skip, Wl1, b1, Wr1, Wl2, b2, Wr2, Wl3, b3, Wr3, a):
    n, d = x.shape
    e = edge_index.shape[1]
    src = edge_index[0]
    dst = edge_index[1]
    zeros = jnp.zeros((n, d), jnp.float32)
    ones = jnp.ones((_K, _CW), jnp.float32)

    seg_cnt = _seg_kernel(n, d, e, with_cnt=True)
    seg = _seg_kernel(n, d, e, with_cnt=False)
    tc1 = _tc_layer(n, d, 1)
    tc2 = _tc_layer(n, d, 2)
    tc3 = _tc_layer(n, d, 3)

    a2 = a.reshape(1, 1)
    b1r = b1.reshape(1, d)
    b2r = b2.reshape(1, d)
    b3r = b3.reshape(1, d)

    p1, cnt = seg_cnt(x, src, dst, zeros, ones)
    z2 = tc1(p1, cnt, x, Wl1, b1r, Wr1, W_skip, a2)
    p2 = seg(z2, src, dst, zeros)
    z3 = tc2(p2, cnt, z2, Wl2, b2r, Wr2, a2)
    p3 = seg(z3, src, dst, zeros)
    h3 = tc3(p3, cnt, z3, Wl3, b3r, Wr3, a2)
    return h3


# speedup vs baseline: 4.5157x; 4.5157x over previous
import jax
import jax.numpy as jnp
from jax import lax
from jax.experimental import pallas as pl
from jax.experimental.pallas import tpu as pltpu
from jax.experimental.pallas import tpu_sc as plsc

_NC = 2
_NS = 16
_NW = _NC * _NS
_K = 80


def _npad(n):
    q = _NS * _K
    return ((n + q - 1) // q) * q


def _seg_kernel(n, d, e, with_gather):
    ew = e // _NW
    nchunk = ew // _K
    npad = _npad(n)
    rt = npad // _NS

    mesh = plsc.VectorSubcoreMesh(core_axis_name="c", subcore_axis_name="s",
                                  num_cores=_NC, num_subcores=_NS)

    def body(*refs):
        if with_gather:
            (h_hbm, src_hbm, dst_hbm, zeros_hbm, out_hbm,
             src_v, dst_v, rows_v, acc_sh, sem) = refs
        else:
            (ones_hbm, dst_hbm, zeros_hbm, out_hbm,
             dst_v, rows_v, acc_sh, sem) = refs
        c = lax.axis_index("c")
        s = lax.axis_index("s")
        wid = c * _NS + s
        r0 = s * rt

        @pl.loop(0, rt // _K)
        def _zero(k):
            rk = r0 + k * _K
            pltpu.sync_copy(zeros_hbm.at[pl.ds(rk, _K)], rows_v)
            pltpu.sync_copy(rows_v, acc_sh.at[pl.ds(rk, _K)])

        if not with_gather:
            pltpu.sync_copy(ones_hbm, rows_v)
        plsc.subcore_barrier()

        base = wid * ew

        @pl.loop(0, nchunk)
        def _chunk(j):
            off = base + j * _K
            if with_gather:
                pltpu.sync_copy(src_hbm.at[pl.ds(off, _K)], src_v)
            pltpu.sync_copy(dst_hbm.at[pl.ds(off, _K)], dst_v)
            if with_gather:
                pltpu.async_copy(h_hbm.at[src_v], rows_v, sem).wait()
            pltpu.sync_copy(rows_v, acc_sh.at[dst_v], add=True)

        plsc.subcore_barrier()

        @pl.loop(0, rt // _K)
        def _writeback(k):
            rk = r0 + k * _K
            pltpu.sync_copy(acc_sh.at[pl.ds(rk, _K)], rows_v)
            pltpu.sync_copy(rows_v, out_hbm.at[c, pl.ds(rk, _K)])

    scratch = [pltpu.VMEM((_K,), jnp.int32)] if with_gather else []
    scratch += [
        pltpu.VMEM((_K,), jnp.int32),
        pltpu.VMEM((_K, d), jnp.float32),
        pltpu.VMEM_SHARED((npad, d), jnp.float32),
        pltpu.SemaphoreType.DMA,
    ]
    return pl.kernel(
        body,
        out_type=[jax.ShapeDtypeStruct((_NC, npad, d), jnp.float32)],
        mesh=mesh, scratch_types=scratch)


def _tc_layer(n, d, mode):
    br = 400
    grid = (n // br,)

    def body(*refs):
        if mode == 1:
            p_ref, cnt_ref, hin_ref, wl_ref, b_ref, wr_ref, wsk_ref, a_ref, out_ref = refs
        else:
            p_ref, cnt_ref, hin_ref, wl_ref, b_ref, wr_ref, a_ref, out_ref = refs
        cnt = cnt_ref[0, :, 0:1] + cnt_ref[1, :, 0:1]
        inv = 1.0 / jnp.maximum(cnt, 1.0)
        agg = (p_ref[0] + p_ref[1]) * inv
        hin = hin_ref[...]
        dn = (((1,), (1,)), ((), ()))
        h = lax.dot_general(agg, wl_ref[...], dn,
                            precision=lax.Precision.HIGHEST,
                            preferred_element_type=jnp.float32)
        h = h + b_ref[...] + lax.dot_general(
            hin, wr_ref[...], dn, precision=lax.Precision.HIGHEST,
            preferred_element_type=jnp.float32)
        av = a_ref[0, 0]
        h = jnp.where(h >= 0, h, av * h)
        if mode == 1:
            h = h + lax.dot_general(hin, wsk_ref[...], dn,
                                    precision=lax.Precision.HIGHEST,
                                    preferred_element_type=jnp.float32)
        elif mode == 2:
            h = h + hin
        out_ref[...] = h

    in_specs = [
        pl.BlockSpec((_NC, br, d), lambda i: (0, i, 0)),
        pl.BlockSpec((_NC, br, d), lambda i: (0, i, 0)),
        pl.BlockSpec((br, d), lambda i: (i, 0)),
        pl.BlockSpec((d, d), lambda i: (0, 0)),
        pl.BlockSpec((1, d), lambda i: (0, 0)),
        pl.BlockSpec((d, d), lambda i: (0, 0)),
    ]
    if mode == 1:
        in_specs.append(pl.BlockSpec((d, d), lambda i: (0, 0)))
    in_specs.append(pl.BlockSpec(memory_space=pltpu.SMEM))

    return pl.pallas_call(
        body,
        grid=grid,
        in_specs=in_specs,
        out_specs=pl.BlockSpec((br, d), lambda i: (i, 0)),
        out_shape=jax.ShapeDtypeStruct((n, d), jnp.float32),
    )


def kernel(x, edge_index, W_skip, Wl1, b1, Wr1, Wl2, b2, Wr2, Wl3, b3, Wr3, a):
    n, d = x.shape
    e = edge_index.shape[1]
    src = edge_index[0]
    dst = edge_index[1]
    zeros = jnp.zeros((_npad(n), d), jnp.float32)
    ones = jnp.ones((_K, d), jnp.float32)

    seg = _seg_kernel(n, d, e, with_gather=True)
    cntk = _seg_kernel(n, d, e, with_gather=False)
    tc1 = _tc_layer(n, d, 1)
    tc2 = _tc_layer(n, d, 2)
    tc3 = _tc_layer(n, d, 3)

    a2 = a.reshape(1, 1)
    b1r = b1.reshape(1, d)
    b2r = b2.reshape(1, d)
    b3r = b3.reshape(1, d)

    cnt, = cntk(ones, dst, zeros)
    p1, = seg(x, src, dst, zeros)
    z2 = tc1(p1, cnt, x, Wl1, b1r, Wr1, W_skip, a2)
    p2, = seg(z2, src, dst, zeros)
    z3 = tc2(p2, cnt, z2, Wl2, b2r, Wr2, a2)
    p3, = seg(z3, src, dst, zeros)
    h3 = tc3(p3, cnt, z3, Wl3, b3r, Wr3, a2)
    return h3


# speedup vs baseline: 6.6138x; 1.4646x over previous
import jax
import jax.numpy as jnp
from jax import lax
from jax.experimental import pallas as pl
from jax.experimental.pallas import tpu as pltpu
from jax.experimental.pallas import tpu_sc as plsc

_NC = 2
_NS = 16
_NW = _NC * _NS
_K = 80


def _npad(n):
    q = _NS * _K
    return ((n + q - 1) // q) * q


def _seg_kernel(n, d, e, with_gather):
    ew = e // _NW
    nchunk = ew // _K
    npad = _npad(n)
    rt = npad // _NS

    mesh = plsc.VectorSubcoreMesh(core_axis_name="c", subcore_axis_name="s",
                                  num_cores=_NC, num_subcores=_NS)

    def body(*refs):
        if with_gather:
            (h_hbm, src_hbm, dst_hbm, zeros_hbm, out_hbm,
             src_v0, src_v1, dst_v0, dst_v1, rows_v0, rows_v1,
             acc_sh, gsem, ssem) = refs
            src_v = (src_v0, src_v1)
            rows_v = (rows_v0, rows_v1)
        else:
            (ones_hbm, dst_hbm, zeros_hbm, out_hbm,
             dst_v0, dst_v1, rows_v0, acc_sh, ssem) = refs
            rows_v = (rows_v0, rows_v0)
        dst_v = (dst_v0, dst_v1)
        c = lax.axis_index("c")
        s = lax.axis_index("s")
        wid = c * _NS + s
        r0 = s * rt

        @pl.loop(0, rt // _K)
        def _zero(k):
            rk = r0 + k * _K
            pltpu.sync_copy(zeros_hbm.at[pl.ds(rk, _K)], rows_v0)
            pltpu.sync_copy(rows_v0, acc_sh.at[pl.ds(rk, _K)])

        if not with_gather:
            pltpu.sync_copy(ones_hbm, rows_v0)
        plsc.subcore_barrier()

        base = wid * ew

        def load_idx(p, j):
            off = base + j * _K
            if with_gather:
                pltpu.sync_copy(src_hbm.at[pl.ds(off, _K)], src_v[p])
            pltpu.sync_copy(dst_hbm.at[pl.ds(off, _K)], dst_v[p])

        def gather_start(p):
            if with_gather:
                pltpu.async_copy(h_hbm.at[src_v[p]], rows_v[p], gsem)

        def gather_wait(p):
            if with_gather:
                pltpu.make_async_copy(
                    h_hbm.at[src_v[p]], rows_v[p], gsem).wait()

        def scatter_start(p):
            pltpu.async_copy(rows_v[p], acc_sh.at[dst_v[p]], ssem, add=True)

        def scatter_wait(p):
            pltpu.make_async_copy(
                rows_v[p], acc_sh.at[dst_v[p]], ssem).wait()

        def stage(j, p):
            scatter_wait(p)
            load_idx(p, j)
            gather_wait(1 - p)
            scatter_start(1 - p)
            gather_start(p)

        load_idx(0, 0)
        gather_start(0)
        load_idx(1, 1)
        gather_wait(0)
        scatter_start(0)
        gather_start(1)

        start = 2
        if (nchunk - 2) % 2 == 1:
            stage(2, 0)
            start = 3
        sp = start & 1

        @pl.loop(0, (nchunk - start) // 2)
        def _steady(k):
            j = start + 2 * k
            stage(j, sp)
            stage(j + 1, 1 - sp)

        pe = (nchunk - 1) & 1
        scatter_wait(1 - pe)
        gather_wait(pe)
        scatter_start(pe)
        scatter_wait(pe)

        plsc.subcore_barrier()

        @pl.loop(0, rt // _K)
        def _writeback(k):
            rk = r0 + k * _K
            pltpu.sync_copy(acc_sh.at[pl.ds(rk, _K)], rows_v0)
            pltpu.sync_copy(rows_v0, out_hbm.at[c, pl.ds(rk, _K)])

    if with_gather:
        scratch = [
            pltpu.VMEM((_K,), jnp.int32), pltpu.VMEM((_K,), jnp.int32),
            pltpu.VMEM((_K,), jnp.int32), pltpu.VMEM((_K,), jnp.int32),
            pltpu.VMEM((_K, d), jnp.float32), pltpu.VMEM((_K, d), jnp.float32),
            pltpu.VMEM_SHARED((npad, d), jnp.float32),
            pltpu.SemaphoreType.DMA, pltpu.SemaphoreType.DMA,
        ]
    else:
        scratch = [
            pltpu.VMEM((_K,), jnp.int32), pltpu.VMEM((_K,), jnp.int32),
            pltpu.VMEM((_K, d), jnp.float32),
            pltpu.VMEM_SHARED((npad, d), jnp.float32),
            pltpu.SemaphoreType.DMA,
        ]
    return pl.kernel(
        body,
        out_type=[jax.ShapeDtypeStruct((_NC, npad, d), jnp.float32)],
        mesh=mesh, scratch_types=scratch)


def _tc_layer(n, d, mode):
    br = 400
    grid = (n // br,)

    def body(*refs):
        if mode == 1:
            p_ref, cnt_ref, hin_ref, wl_ref, b_ref, wr_ref, wsk_ref, a_ref, out_ref = refs
        else:
            p_ref, cnt_ref, hin_ref, wl_ref, b_ref, wr_ref, a_ref, out_ref = refs
        cnt = cnt_ref[0, :, 0:1] + cnt_ref[1, :, 0:1]
        inv = 1.0 / jnp.maximum(cnt, 1.0)
        agg = (p_ref[0] + p_ref[1]) * inv
        hin = hin_ref[...]
        dn = (((1,), (1,)), ((), ()))
        h = lax.dot_general(agg, wl_ref[...], dn,
                            precision=lax.Precision.HIGHEST,
                            preferred_element_type=jnp.float32)
        h = h + b_ref[...] + lax.dot_general(
            hin, wr_ref[...], dn, precision=lax.Precision.HIGHEST,
            preferred_element_type=jnp.float32)
        av = a_ref[0, 0]
        h = jnp.where(h >= 0, h, av * h)
        if mode == 1:
            h = h + lax.dot_general(hin, wsk_ref[...], dn,
                                    precision=lax.Precision.HIGHEST,
                                    preferred_element_type=jnp.float32)
        elif mode == 2:
            h = h + hin
        out_ref[...] = h

    in_specs = [
        pl.BlockSpec((_NC, br, d), lambda i: (0, i, 0)),
        pl.BlockSpec((_NC, br, d), lambda i: (0, i, 0)),
        pl.BlockSpec((br, d), lambda i: (i, 0)),
        pl.BlockSpec((d, d), lambda i: (0, 0)),
        pl.BlockSpec((1, d), lambda i: (0, 0)),
        pl.BlockSpec((d, d), lambda i: (0, 0)),
    ]
    if mode == 1:
        in_specs.append(pl.BlockSpec((d, d), lambda i: (0, 0)))
    in_specs.append(pl.BlockSpec(memory_space=pltpu.SMEM))

    return pl.pallas_call(
        body,
        grid=grid,
        in_specs=in_specs,
        out_specs=pl.BlockSpec((br, d), lambda i: (i, 0)),
        out_shape=jax.ShapeDtypeStruct((n, d), jnp.float32),
    )


def kernel(x, edge_index, W_skip, Wl1, b1, Wr1, Wl2, b2, Wr2, Wl3, b3, Wr3, a):
    n, d = x.shape
    e = edge_index.shape[1]
    src = edge_index[0]
    dst = edge_index[1]
    zeros = jnp.zeros((_npad(n), d), jnp.float32)
    ones = jnp.ones((_K, d), jnp.float32)

    seg = _seg_kernel(n, d, e, with_gather=True)
    cntk = _seg_kernel(n, d, e, with_gather=False)
    tc1 = _tc_layer(n, d, 1)
    tc2 = _tc_layer(n, d, 2)
    tc3 = _tc_layer(n, d, 3)

    a2 = a.reshape(1, 1)
    b1r = b1.reshape(1, d)
    b2r = b2.reshape(1, d)
    b3r = b3.reshape(1, d)

    cnt, = cntk(ones, dst, zeros)
    p1, = seg(x, src, dst, zeros)
    z2 = tc1(p1, cnt, x, Wl1, b1r, Wr1, W_skip, a2)
    p2, = seg(z2, src, dst, zeros)
    z3 = tc2(p2, cnt, z2, Wl2, b2r, Wr2, a2)
    p3, = seg(z3, src, dst, zeros)
    h3 = tc3(p3, cnt, z3, Wl3, b3r, Wr3, a2)
    return h3


# speedup vs baseline: 7.0244x; 1.0621x over previous
import jax
import jax.numpy as jnp
from jax import lax
from jax.experimental import pallas as pl
from jax.experimental.pallas import tpu as pltpu
from jax.experimental.pallas import tpu_sc as plsc

_NC = 2
_NS = 16
_NW = _NC * _NS
_K = 80


def _npad(n):
    q = _NS * _K
    return ((n + q - 1) // q) * q


def _seg_kernel(n, d, e, with_gather, nchunk_pad=None):
    ew = e // _NW
    nchunk = ew // _K if nchunk_pad is None else nchunk_pad
    npad = _npad(n)
    rt = npad // _NS

    mesh = plsc.VectorSubcoreMesh(core_axis_name="c", subcore_axis_name="s",
                                  num_cores=_NC, num_subcores=_NS)

    def body(*refs):
        if with_gather:
            (h_hbm, src_hbm, dst_hbm, zeros_hbm, out_hbm,
             src_v0, src_v1, dst_v0, dst_v1,
             rows_v0, rows_v1, acc_sh, gsem, ssem) = refs
            rows_v = (rows_v0, rows_v1)
            src_v = (src_v0, src_v1)
        else:
            (ones_hbm, dst_hbm, zeros_hbm, out_hbm,
             dst_all, dst_v0, dst_v1, rows_v0, acc_sh, ssem) = refs
            rows_v = (rows_v0, rows_v0)
        dst_v = (dst_v0, dst_v1)
        c = lax.axis_index("c")
        s = lax.axis_index("s")
        wid = c * _NS + s
        r0 = s * rt

        @pl.loop(0, rt // _K)
        def _zero(k):
            rk = r0 + k * _K
            pltpu.sync_copy(zeros_hbm.at[pl.ds(rk, _K)], rows_v0)
            pltpu.sync_copy(rows_v0, acc_sh.at[pl.ds(rk, _K)])

        if not with_gather:
            pltpu.sync_copy(dst_hbm.at[wid], dst_all)
            pltpu.sync_copy(ones_hbm, rows_v0)
        plsc.subcore_barrier()

        base = wid * ew

        def load_idx(p, j):
            if with_gather:
                off = base + j * _K
                pltpu.sync_copy(src_hbm.at[pl.ds(off, _K)], src_v[p])
                pltpu.sync_copy(dst_hbm.at[pl.ds(off, _K)], dst_v[p])
            else:
                for i in range(_K // 16):
                    sl = pl.ds(i * 16, 16)
                    dst_v[p][sl] = dst_all[j, sl]

        def gather_start(p, j):
            if with_gather:
                pltpu.async_copy(h_hbm.at[src_v[p]], rows_v[p], gsem)

        def gather_wait(p, j):
            if with_gather:
                pltpu.make_async_copy(
                    h_hbm.at[src_v[p]], rows_v[p], gsem).wait()

        def scatter_start(p, j):
            pltpu.async_copy(rows_v[p], acc_sh.at[dst_v[p]], ssem,
                             add=True)

        def scatter_wait(p, j):
            pltpu.make_async_copy(
                rows_v[p], acc_sh.at[dst_v[p]], ssem).wait()

        if with_gather:
            def stage(j, p):
                scatter_wait(p, j)
                load_idx(p, j)
                gather_wait(1 - p, j)
                scatter_start(1 - p, j - 1)
                gather_start(p, j)

            load_idx(0, 0)
            gather_start(0, 0)
            load_idx(1, 1)
            gather_wait(0, 0)
            scatter_start(0, 0)
            gather_start(1, 1)

            start = 2
            if (nchunk - 2) % 2 == 1:
                stage(2, 0)
                start = 3
            sp = start & 1

            @pl.loop(0, (nchunk - start) // 2)
            def _steady(k):
                j = start + 2 * k
                stage(j, sp)
                stage(j + 1, 1 - sp)

            pe = (nchunk - 1) & 1
            scatter_wait(1 - pe, 0)
            gather_wait(pe, nchunk - 1)
            scatter_start(pe, nchunk - 1)
            scatter_wait(pe, 0)
        else:
            def cstage(j, p):
                scatter_wait(p, j)
                load_idx(p, j)
                scatter_start(p, j)

            load_idx(0, 0)
            scatter_start(0, 0)
            load_idx(1, 1)
            scatter_start(1, 1)

            cstart = 2
            if (nchunk - 2) % 2 == 1:
                cstage(2, 0)
                cstart = 3
            csp = cstart & 1

            @pl.loop(0, (nchunk - cstart) // 2)
            def _cnt(k):
                j = cstart + 2 * k
                cstage(j, csp)
                cstage(j + 1, 1 - csp)

            cpe = (nchunk - 1) & 1
            scatter_wait(1 - cpe, 0)
            scatter_wait(cpe, 0)

        plsc.subcore_barrier()

        @pl.loop(0, rt // _K)
        def _writeback(k):
            rk = r0 + k * _K
            pltpu.sync_copy(acc_sh.at[pl.ds(rk, _K)], rows_v0)
            pltpu.sync_copy(rows_v0, out_hbm.at[c, pl.ds(rk, _K)])

    if with_gather:
        scratch = [
            pltpu.VMEM((_K,), jnp.int32), pltpu.VMEM((_K,), jnp.int32),
            pltpu.VMEM((_K,), jnp.int32), pltpu.VMEM((_K,), jnp.int32),
            pltpu.VMEM((_K, d), jnp.float32), pltpu.VMEM((_K, d), jnp.float32),
            pltpu.VMEM_SHARED((npad, d), jnp.float32),
            pltpu.SemaphoreType.DMA, pltpu.SemaphoreType.DMA,
        ]
    else:
        scratch = [
            pltpu.VMEM((nchunk, _K), jnp.int32),
            pltpu.VMEM((_K,), jnp.int32), pltpu.VMEM((_K,), jnp.int32),
            pltpu.VMEM((_K, d), jnp.float32),
            pltpu.VMEM_SHARED((npad, d), jnp.float32),
            pltpu.SemaphoreType.DMA,
        ]
    return pl.kernel(
        body,
        out_type=[jax.ShapeDtypeStruct((_NC, npad, d), jnp.float32)],
        mesh=mesh, scratch_types=scratch)


def _tc_layer(n, d, mode):
    br = 400
    grid = (n // br,)

    def body(*refs):
        if mode == 1:
            p_ref, cnt_ref, hin_ref, wl_ref, b_ref, wr_ref, wsk_ref, a_ref, out_ref = refs
        else:
            p_ref, cnt_ref, hin_ref, wl_ref, b_ref, wr_ref, a_ref, out_ref = refs
        cnt = cnt_ref[0, :, 0:1] + cnt_ref[1, :, 0:1]
        inv = 1.0 / jnp.maximum(cnt, 1.0)
        agg = (p_ref[0] + p_ref[1]) * inv
        hin = hin_ref[...]
        dn = (((1,), (1,)), ((), ()))
        h = lax.dot_general(agg, wl_ref[...], dn,
                            precision=lax.Precision.HIGHEST,
                            preferred_element_type=jnp.float32)
        h = h + b_ref[...] + lax.dot_general(
            hin, wr_ref[...], dn, precision=lax.Precision.HIGHEST,
            preferred_element_type=jnp.float32)
        av = a_ref[0, 0]
        h = jnp.where(h >= 0, h, av * h)
        if mode == 1:
            h = h + lax.dot_general(hin, wsk_ref[...], dn,
                                    precision=lax.Precision.HIGHEST,
                                    preferred_element_type=jnp.float32)
        elif mode == 2:
            h = h + hin
        out_ref[...] = h

    in_specs = [
        pl.BlockSpec((_NC, br, d), lambda i: (0, i, 0)),
        pl.BlockSpec((_NC, br, d), lambda i: (0, i, 0)),
        pl.BlockSpec((br, d), lambda i: (i, 0)),
        pl.BlockSpec((d, d), lambda i: (0, 0)),
        pl.BlockSpec((1, d), lambda i: (0, 0)),
        pl.BlockSpec((d, d), lambda i: (0, 0)),
    ]
    if mode == 1:
        in_specs.append(pl.BlockSpec((d, d), lambda i: (0, 0)))
    in_specs.append(pl.BlockSpec(memory_space=pltpu.SMEM))

    return pl.pallas_call(
        body,
        grid=grid,
        in_specs=in_specs,
        out_specs=pl.BlockSpec((br, d), lambda i: (i, 0)),
        out_shape=jax.ShapeDtypeStruct((n, d), jnp.float32),
    )


def kernel(x, edge_index, W_skip, Wl1, b1, Wr1, Wl2, b2, Wr2, Wl3, b3, Wr3, a):
    n, d = x.shape
    e = edge_index.shape[1]
    nchunk = e // (_NW * _K)
    npadn = _npad(n)
    src = edge_index[0]
    dst = edge_index[1]
    nck = nchunk + (nchunk % 2)
    dst3 = dst.reshape(_NW, nchunk, _K)
    if nck != nchunk:
        dst3 = jnp.concatenate(
            [dst3, jnp.full((_NW, nck - nchunk, _K), npadn - 1, jnp.int32)],
            axis=1)
    zeros = jnp.zeros((npadn, d), jnp.float32)
    ones = jnp.ones((_K, d), jnp.float32)

    seg = _seg_kernel(n, d, e, with_gather=True)
    cntk = _seg_kernel(n, d, e, with_gather=False, nchunk_pad=nck)
    tc1 = _tc_layer(n, d, 1)
    tc2 = _tc_layer(n, d, 2)
    tc3 = _tc_layer(n, d, 3)

    a2 = a.reshape(1, 1)
    b1r = b1.reshape(1, d)
    b2r = b2.reshape(1, d)
    b3r = b3.reshape(1, d)

    cnt, = cntk(ones, dst3, zeros)
    p1, = seg(x, src, dst, zeros)
    z2 = tc1(p1, cnt, x, Wl1, b1r, Wr1, W_skip, a2)
    p2, = seg(z2, src, dst, zeros)
    z3 = tc2(p2, cnt, z2, Wl2, b2r, Wr2, a2)
    p3, = seg(z3, src, dst, zeros)
    h3 = tc3(p3, cnt, z3, Wl3, b3r, Wr3, a2)
    return h3
